# Initial kernel scaffold; baseline (speedup 1.0000x reference)
import jax
import jax.numpy as jnp
from jax.experimental import pallas as pl


def kernel(input_features, neighbor_inds, weightnet, additional_features):
    raise NotImplementedError("write your pallas kernel here")


# speedup vs baseline: 3.2874x; 3.2874x over previous
import functools

import jax
import jax.numpy as jnp
from jax import lax
from jax.experimental import pallas as pl
from jax.experimental.pallas import tpu as pltpu
from jax.experimental.pallas import tpu_sc as plsc

N = 10000
K = 32
C_IN = 128
C_ADD = 16
C_MID = 16
C_TOT = C_IN + C_ADD

_NC, _NS = 2, 16
_NW = _NC * _NS
_ROWS = N * K
_ROWS_PER_W = _ROWS // _NW
_CHUNK = 80
_NCHUNK = _ROWS_PER_W // _CHUNK


def _sc_gather_body(table_hbm, idx_hbm, out_hbm, idx_v, rows_v, sem):
    wid = lax.axis_index("s") * _NC + lax.axis_index("c")
    base = wid * _ROWS_PER_W

    def body(j, carry):
        off = base + j * _CHUNK
        pltpu.sync_copy(idx_hbm.at[pl.ds(off, _CHUNK)], idx_v)
        pltpu.async_copy(table_hbm.at[idx_v], rows_v, sem).wait()
        pltpu.sync_copy(rows_v, out_hbm.at[pl.ds(off, _CHUNK)])
        return carry

    lax.fori_loop(0, _NCHUNK, body, 0)


def _sc_gather(table, idx_flat):
    mesh = plsc.VectorSubcoreMesh(core_axis_name="c", subcore_axis_name="s")
    fn = functools.partial(
        pl.kernel,
        out_type=jax.ShapeDtypeStruct((_ROWS, C_IN), jnp.float32),
        mesh=mesh,
        scratch_types=[
            pltpu.VMEM((_CHUNK,), jnp.int32),
            pltpu.VMEM((_CHUNK, C_IN), jnp.float32),
            pltpu.SemaphoreType.DMA,
        ],
    )(_sc_gather_body)
    return fn(table, idx_flat)


_PB = 80


def _tc_body(g_ref, w_ref, a_ref, o_ref):
    wt = w_ref[...]
    r1 = lax.dot_general(wt, g_ref[...], (((2,), (1,)), ((0,), (0,))),
                         preferred_element_type=jnp.float32)
    r2 = lax.dot_general(wt, a_ref[...], (((2,), (1,)), ((0,), (0,))),
                         preferred_element_type=jnp.float32)
    o_ref[:, 0:C_IN] = r1.reshape(_PB * C_MID, C_IN)
    o_ref[:, C_IN:C_TOT] = r2.reshape(_PB * C_MID, C_ADD)


def _tc_compute(gathered, weightnet_t, additional):
    grid = N // _PB
    return pl.pallas_call(
        _tc_body,
        grid=(grid,),
        in_specs=[
            pl.BlockSpec((_PB, K, C_IN), lambda i: (i, 0, 0)),
            pl.BlockSpec((_PB, C_MID, K), lambda i: (i, 0, 0)),
            pl.BlockSpec((_PB, K, C_ADD), lambda i: (i, 0, 0)),
        ],
        out_specs=pl.BlockSpec((_PB * C_MID, C_TOT), lambda i: (i, 0)),
        out_shape=jax.ShapeDtypeStruct((N * C_MID, C_TOT), jnp.float32),
    )(gathered, weightnet_t, additional)


def kernel(input_features, neighbor_inds, weightnet, additional_features):
    table = input_features[0]
    idx_flat = neighbor_inds[0].astype(jnp.int32).reshape(_ROWS)
    gathered = _sc_gather(table, idx_flat).reshape(N, K, C_IN)
    w_t = weightnet[0].transpose(0, 2, 1)
    out_t = _tc_compute(gathered, w_t, additional_features[0])
    out = out_t.reshape(N, C_MID, C_TOT).transpose(0, 2, 1).reshape(1, N, C_TOT * C_MID)
    return out
